# baseline (speedup 1.0000x reference)
import functools

import jax
import jax.numpy as jnp
from jax import lax
from jax.experimental import pallas as pl
from jax.experimental.pallas import tpu as pltpu
from jax.experimental.pallas import tpu_sc as plsc

VOCAB = 1000000
DIM = 64
WIDE = 128
BATCH = 16384
NUM_NEG = 10
NUM_ROWS = NUM_NEG + 2
NUM_CN = NUM_NEG + 1
PAR_ROWS = 16

NC = 2
NS = 16
NW = NC * NS
SUB = 256

TL = 2048
NWB = (VOCAB + TL - 1) // TL
WROWS = NWB * (TL // 2)


def _relayout_kernel(t_ref, o_ref):
    blk = t_ref[...]
    row = lax.broadcasted_iota(jnp.int32, (DIM, DIM), 0)
    col = lax.broadcasted_iota(jnp.int32, (DIM, DIM), 1)
    eye = (row == col).astype(jnp.float32)
    dn = (((0,), (0,)), ((), ()))
    o_ref[:, :DIM] = lax.dot_general(
        blk[:, : TL // 2], eye, dn, preferred_element_type=jnp.float32)
    o_ref[:, DIM:] = lax.dot_general(
        blk[:, TL // 2:], eye, dn, preferred_element_type=jnp.float32)


def _relayout(table_t):
    return pl.pallas_call(
        _relayout_kernel,
        grid=(NWB,),
        in_specs=[pl.BlockSpec((DIM, TL), lambda i: (0, i))],
        out_specs=pl.BlockSpec((TL // 2, WIDE), lambda i: (i, 0)),
        out_shape=jax.ShapeDtypeStruct((WROWS, WIDE), jnp.float32),
        compiler_params=pltpu.CompilerParams(
            dimension_semantics=("parallel",)),
    )(table_t)


def _sc_gather(table_wide, idx_flat, n_rows):
    per_w = n_rows * BATCH // NW
    nchunk = per_w // SUB
    chunk_b = BATCH // NW
    mesh = plsc.VectorSubcoreMesh(core_axis_name="c", subcore_axis_name="s")

    @functools.partial(
        pl.kernel,
        mesh=mesh,
        out_type=jax.ShapeDtypeStruct((n_rows * BATCH, WIDE), jnp.float32),
        scratch_types=[
            pltpu.VMEM((per_w,), jnp.int32),
            pltpu.VMEM((SUB, WIDE), jnp.float32),
            pltpu.VMEM((SUB, WIDE), jnp.float32),
            pltpu.SemaphoreType.DMA,
            pltpu.SemaphoreType.DMA,
        ],
    )
    def gather_kernel(t_hbm, idx_hbm, g_hbm, idx_v, rows_a, rows_b,
                      sem_a, sem_b):
        wid = lax.axis_index("s") * NC + lax.axis_index("c")
        pltpu.sync_copy(idx_hbm.at[pl.ds(wid * per_w, per_w)], idx_v)
        bufs = (rows_a, rows_b)
        sems = (sem_a, sem_b)
        pend = [None, None]

        def dst(c):
            j, h = divmod(c, chunk_b // SUB)
            return j * BATCH + wid * chunk_b + h * SUB

        for c in range(nchunk):
            s = c % 2
            pend[s] = pltpu.async_copy(
                t_hbm.at[idx_v.at[pl.ds(c * SUB, SUB)]], bufs[s], sems[s]
            )
            if c > 0:
                pend[1 - s].wait()
                pltpu.sync_copy(
                    bufs[1 - s], g_hbm.at[pl.ds(dst(c - 1), SUB)]
                )
        s = (nchunk - 1) % 2
        pend[s].wait()
        pltpu.sync_copy(bufs[s], g_hbm.at[pl.ds(dst(nchunk - 1), SUB)])

    return gather_kernel(table_wide, idx_flat)


BB = 1024


def _loss_kernel(c_ref, n_ref, p_ref, o_ref):
    i = pl.program_id(1)
    c = c_ref[...]
    n = n_ref[...]
    par = p_ref[...]
    csel = jnp.where(par[0][:, None] > 0.5, c[:, DIM:], c[:, :DIM])
    nsel = jnp.where(par[1:NUM_ROWS][:, :, None] > 0.5,
                     n[:, :, DIM:], n[:, :, :DIM])
    scores = jnp.sum(csel[None, :, :] * nsel, axis=-1)

    def softplus(x):
        return jnp.maximum(x, 0.0) + jnp.log1p(jnp.exp(-jnp.abs(x)))

    block = jnp.sum(softplus(-scores[0])) + jnp.sum(softplus(scores[1:]))

    @pl.when(i == 0)
    def _():
        o_ref[...] = jnp.zeros_like(o_ref)

    o_ref[...] += block


def _tc_loss(g_center, g_cn, parity):
    n3 = g_cn.reshape(NUM_CN, BATCH, WIDE)
    nb2 = BATCH // BB // 2
    out = pl.pallas_call(
        _loss_kernel,
        grid=(2, nb2),
        in_specs=[
            pl.BlockSpec((BB, WIDE), lambda c, j: (c * nb2 + j, 0)),
            pl.BlockSpec((NUM_CN, BB, WIDE), lambda c, j: (0, c * nb2 + j, 0)),
            pl.BlockSpec((PAR_ROWS, BB), lambda c, j: (0, c * nb2 + j)),
        ],
        out_specs=pl.BlockSpec((1, 1, 1), lambda c, j: (c, 0, 0)),
        out_shape=jax.ShapeDtypeStruct((2, 1, 1), jnp.float32),
        compiler_params=pltpu.CompilerParams(
            dimension_semantics=("parallel", "arbitrary")),
    )(g_center, n3, parity)
    return (out[0, 0, 0] + out[1, 0, 0]) / BATCH


def kernel(center_nodes, context_nodes, negative_nodes, input_emb, output_emb):
    idx = jnp.concatenate(
        [
            center_nodes.astype(jnp.int32)[None, :],
            context_nodes.astype(jnp.int32)[None, :],
            negative_nodes.astype(jnp.int32).T,
        ],
        axis=0,
    )
    r = idx % TL
    sel = (r >= TL // 2).astype(jnp.float32)
    parity = jnp.zeros((PAR_ROWS, BATCH), jnp.float32)
    parity = parity.at[:NUM_ROWS].set(sel)
    half = (idx // TL) * (TL // 2) + (r % (TL // 2))
    idx_cn = (
        half[1:].reshape(NUM_CN, NW, BATCH // NW)
        .transpose(1, 0, 2).reshape(-1)
    )
    idx_c = half[0].reshape(NW, BATCH // NW).reshape(-1)
    out_wide = _relayout(output_emb.T)
    g_cn = _sc_gather(out_wide, idx_cn, NUM_CN)
    in_wide = _relayout(input_emb.T)
    g_center = _sc_gather(in_wide, idx_c, 1)
    return _tc_loss(g_center, g_cn, parity)


# speedup vs baseline: 1.5495x; 1.5495x over previous
import functools

import jax
import jax.numpy as jnp
from jax import lax
from jax.experimental import pallas as pl
from jax.experimental.pallas import tpu as pltpu
from jax.experimental.pallas import tpu_sc as plsc

VOCAB = 1000000
DIM = 64
WIDE = 128
BATCH = 16384
NUM_NEG = 10
NUM_ROWS = NUM_NEG + 2
NUM_CN = NUM_NEG + 1
PAR_ROWS = 16

NC = 2
NS = 16
NW = NC * NS
SUB = 256

TL = 8192
NWB = (VOCAB + TL - 1) // TL
WROWS = NWB * (TL // 2)


def _relayout_kernel(t_ref, o_ref):
    blk = t_ref[...]
    row = lax.broadcasted_iota(jnp.int32, (DIM, DIM), 0)
    col = lax.broadcasted_iota(jnp.int32, (DIM, DIM), 1)
    eye = (row == col).astype(jnp.float32)
    dn = (((0,), (0,)), ((), ()))
    o_ref[:, :DIM] = lax.dot_general(
        blk[:, : TL // 2], eye, dn, preferred_element_type=jnp.float32)
    o_ref[:, DIM:] = lax.dot_general(
        blk[:, TL // 2:], eye, dn, preferred_element_type=jnp.float32)


def _relayout(table_t):
    return pl.pallas_call(
        _relayout_kernel,
        grid=(NWB,),
        in_specs=[pl.BlockSpec((DIM, TL), lambda i: (0, i))],
        out_specs=pl.BlockSpec((TL // 2, WIDE), lambda i: (i, 0)),
        out_shape=jax.ShapeDtypeStruct((WROWS, WIDE), jnp.float32),
        compiler_params=pltpu.CompilerParams(
            dimension_semantics=("parallel",)),
    )(table_t)


def _sc_gather(table_wide, idx_flat, n_rows):
    per_w = n_rows * BATCH // NW
    nchunk = per_w // SUB
    chunk_b = BATCH // NW
    mesh = plsc.VectorSubcoreMesh(core_axis_name="c", subcore_axis_name="s")

    @functools.partial(
        pl.kernel,
        mesh=mesh,
        out_type=jax.ShapeDtypeStruct((n_rows * BATCH, WIDE), jnp.float32),
        scratch_types=[
            pltpu.VMEM((per_w,), jnp.int32),
            pltpu.VMEM((SUB, WIDE), jnp.float32),
            pltpu.VMEM((SUB, WIDE), jnp.float32),
            pltpu.SemaphoreType.DMA,
            pltpu.SemaphoreType.DMA,
        ],
    )
    def gather_kernel(t_hbm, idx_hbm, g_hbm, idx_v, rows_a, rows_b,
                      sem_a, sem_b):
        wid = lax.axis_index("s") * NC + lax.axis_index("c")
        pltpu.sync_copy(idx_hbm.at[pl.ds(wid * per_w, per_w)], idx_v)
        bufs = (rows_a, rows_b)
        sems = (sem_a, sem_b)
        pend = [None, None]

        def dst(c):
            j, h = divmod(c, chunk_b // SUB)
            return j * BATCH + wid * chunk_b + h * SUB

        for c in range(nchunk):
            s = c % 2
            pend[s] = pltpu.async_copy(
                t_hbm.at[idx_v.at[pl.ds(c * SUB, SUB)]], bufs[s], sems[s]
            )
            if c > 0:
                pend[1 - s].wait()
                pltpu.sync_copy(
                    bufs[1 - s], g_hbm.at[pl.ds(dst(c - 1), SUB)]
                )
        s = (nchunk - 1) % 2
        pend[s].wait()
        pltpu.sync_copy(bufs[s], g_hbm.at[pl.ds(dst(nchunk - 1), SUB)])

    return gather_kernel(table_wide, idx_flat)


BB = 1024


def _loss_kernel(c_ref, n_ref, p_ref, o_ref):
    i = pl.program_id(1)
    c = c_ref[...]
    n = n_ref[...]
    par = p_ref[...]
    csel = jnp.where(par[0][:, None] > 0.5, c[:, DIM:], c[:, :DIM])
    nsel = jnp.where(par[1:NUM_ROWS][:, :, None] > 0.5,
                     n[:, :, DIM:], n[:, :, :DIM])
    scores = jnp.sum(csel[None, :, :] * nsel, axis=-1)

    def softplus(x):
        return jnp.maximum(x, 0.0) + jnp.log1p(jnp.exp(-jnp.abs(x)))

    block = jnp.sum(softplus(-scores[0])) + jnp.sum(softplus(scores[1:]))

    @pl.when(i == 0)
    def _():
        o_ref[...] = jnp.zeros_like(o_ref)

    o_ref[...] += block


def _tc_loss(g_center, g_cn, parity):
    n3 = g_cn.reshape(NUM_CN, BATCH, WIDE)
    nb2 = BATCH // BB // 2
    out = pl.pallas_call(
        _loss_kernel,
        grid=(2, nb2),
        in_specs=[
            pl.BlockSpec((BB, WIDE), lambda c, j: (c * nb2 + j, 0)),
            pl.BlockSpec((NUM_CN, BB, WIDE), lambda c, j: (0, c * nb2 + j, 0)),
            pl.BlockSpec((PAR_ROWS, BB), lambda c, j: (0, c * nb2 + j)),
        ],
        out_specs=pl.BlockSpec((1, 1, 1), lambda c, j: (c, 0, 0)),
        out_shape=jax.ShapeDtypeStruct((2, 1, 1), jnp.float32),
        compiler_params=pltpu.CompilerParams(
            dimension_semantics=("parallel", "arbitrary")),
    )(g_center, n3, parity)
    return (out[0, 0, 0] + out[1, 0, 0]) / BATCH


def kernel(center_nodes, context_nodes, negative_nodes, input_emb, output_emb):
    idx = jnp.concatenate(
        [
            center_nodes.astype(jnp.int32)[None, :],
            context_nodes.astype(jnp.int32)[None, :],
            negative_nodes.astype(jnp.int32).T,
        ],
        axis=0,
    )
    r = idx % TL
    sel = (r >= TL // 2).astype(jnp.float32)
    parity = jnp.zeros((PAR_ROWS, BATCH), jnp.float32)
    parity = parity.at[:NUM_ROWS].set(sel)
    half = (idx // TL) * (TL // 2) + (r % (TL // 2))
    idx_cn = (
        half[1:].reshape(NUM_CN, NW, BATCH // NW)
        .transpose(1, 0, 2).reshape(-1)
    )
    idx_c = half[0].reshape(NW, BATCH // NW).reshape(-1)
    out_wide = _relayout(output_emb.T)
    g_cn = _sc_gather(out_wide, idx_cn, NUM_CN)
    in_wide = _relayout(input_emb.T)
    g_center = _sc_gather(in_wide, idx_c, 1)
    return _tc_loss(g_center, g_cn, parity)


# speedup vs baseline: 1.7081x; 1.1024x over previous
import functools

import jax
import jax.numpy as jnp
from jax import lax
from jax.experimental import pallas as pl
from jax.experimental.pallas import tpu as pltpu
from jax.experimental.pallas import tpu_sc as plsc

VOCAB = 1000000
DIM = 64
WIDE = 128
BATCH = 16384
NUM_NEG = 10
NUM_ROWS = NUM_NEG + 2
NUM_CN = NUM_NEG + 1
PAR_ROWS = 16

NC = 2
NS = 16
NW = NC * NS
SUB = 256

TL = 16384
NWB = (VOCAB + TL - 1) // TL
WROWS = NWB * (TL // 2)


def _relayout_kernel(t_ref, o_ref):
    blk = t_ref[...]
    row = lax.broadcasted_iota(jnp.int32, (DIM, DIM), 0)
    col = lax.broadcasted_iota(jnp.int32, (DIM, DIM), 1)
    eye = (row == col).astype(jnp.float32)
    dn = (((0,), (0,)), ((), ()))
    o_ref[:, :DIM] = lax.dot_general(
        blk[:, : TL // 2], eye, dn, preferred_element_type=jnp.float32)
    o_ref[:, DIM:] = lax.dot_general(
        blk[:, TL // 2:], eye, dn, preferred_element_type=jnp.float32)


def _relayout(table_t):
    return pl.pallas_call(
        _relayout_kernel,
        grid=(NWB,),
        in_specs=[pl.BlockSpec((DIM, TL), lambda i: (0, i))],
        out_specs=pl.BlockSpec((TL // 2, WIDE), lambda i: (i, 0)),
        out_shape=jax.ShapeDtypeStruct((WROWS, WIDE), jnp.float32),
        compiler_params=pltpu.CompilerParams(
            dimension_semantics=("parallel",)),
    )(table_t)


def _sc_gather(table_wide, idx_flat, n_rows):
    per_w = n_rows * BATCH // NW
    nchunk = per_w // SUB
    chunk_b = BATCH // NW
    mesh = plsc.VectorSubcoreMesh(core_axis_name="c", subcore_axis_name="s")

    @functools.partial(
        pl.kernel,
        mesh=mesh,
        out_type=jax.ShapeDtypeStruct((n_rows * BATCH, WIDE), jnp.float32),
        scratch_types=[
            pltpu.VMEM((per_w,), jnp.int32),
            pltpu.VMEM((SUB, WIDE), jnp.float32),
            pltpu.VMEM((SUB, WIDE), jnp.float32),
            pltpu.SemaphoreType.DMA,
            pltpu.SemaphoreType.DMA,
        ],
    )
    def gather_kernel(t_hbm, idx_hbm, g_hbm, idx_v, rows_a, rows_b,
                      sem_a, sem_b):
        wid = lax.axis_index("s") * NC + lax.axis_index("c")
        pltpu.sync_copy(idx_hbm.at[pl.ds(wid * per_w, per_w)], idx_v)
        bufs = (rows_a, rows_b)
        sems = (sem_a, sem_b)
        pend = [None, None]

        def dst(c):
            j, h = divmod(c, chunk_b // SUB)
            return j * BATCH + wid * chunk_b + h * SUB

        for c in range(nchunk):
            s = c % 2
            pend[s] = pltpu.async_copy(
                t_hbm.at[idx_v.at[pl.ds(c * SUB, SUB)]], bufs[s], sems[s]
            )
            if c > 0:
                pend[1 - s].wait()
                pltpu.sync_copy(
                    bufs[1 - s], g_hbm.at[pl.ds(dst(c - 1), SUB)]
                )
        s = (nchunk - 1) % 2
        pend[s].wait()
        pltpu.sync_copy(bufs[s], g_hbm.at[pl.ds(dst(nchunk - 1), SUB)])

    return gather_kernel(table_wide, idx_flat)


BB = 1024


def _loss_kernel(c_ref, n_ref, p_ref, o_ref):
    i = pl.program_id(1)
    c = c_ref[...]
    n = n_ref[...]
    par = p_ref[...]
    csel = jnp.where(par[0][:, None] > 0.5, c[:, DIM:], c[:, :DIM])
    csel2 = jnp.concatenate([csel, csel], axis=-1)
    prod = csel2[None, :, :] * n
    lane = lax.broadcasted_iota(jnp.int32, (WIDE, 8), 0)
    col = lax.broadcasted_iota(jnp.int32, (WIDE, 8), 1)
    mask = ((col == (lane // DIM)) & (col < 2)).astype(jnp.float32)
    s2 = lax.dot_general(
        prod.reshape(NUM_CN * BB, WIDE), mask,
        (((1,), (0,)), ((), ())), preferred_element_type=jnp.float32,
    ).reshape(NUM_CN, BB, 8)
    s_lo, s_hi = s2[:, :, 0], s2[:, :, 1]
    pn = par[1:NUM_ROWS]
    scores = s_lo + pn * (s_hi - s_lo)

    def softplus(x):
        return jnp.maximum(x, 0.0) + jnp.log1p(jnp.exp(-jnp.abs(x)))

    block = jnp.sum(softplus(-scores[0])) + jnp.sum(softplus(scores[1:]))

    @pl.when(i == 0)
    def _():
        o_ref[...] = jnp.zeros_like(o_ref)

    o_ref[...] += block


def _tc_loss(g_center, g_cn, parity):
    n3 = g_cn.reshape(NUM_CN, BATCH, WIDE)
    nb2 = BATCH // BB // 2
    out = pl.pallas_call(
        _loss_kernel,
        grid=(2, nb2),
        in_specs=[
            pl.BlockSpec((BB, WIDE), lambda c, j: (c * nb2 + j, 0)),
            pl.BlockSpec((NUM_CN, BB, WIDE), lambda c, j: (0, c * nb2 + j, 0)),
            pl.BlockSpec((PAR_ROWS, BB), lambda c, j: (0, c * nb2 + j)),
        ],
        out_specs=pl.BlockSpec((1, 1, 1), lambda c, j: (c, 0, 0)),
        out_shape=jax.ShapeDtypeStruct((2, 1, 1), jnp.float32),
        compiler_params=pltpu.CompilerParams(
            dimension_semantics=("parallel", "arbitrary")),
    )(g_center, n3, parity)
    return (out[0, 0, 0] + out[1, 0, 0]) / BATCH


def kernel(center_nodes, context_nodes, negative_nodes, input_emb, output_emb):
    idx = jnp.concatenate(
        [
            center_nodes.astype(jnp.int32)[None, :],
            context_nodes.astype(jnp.int32)[None, :],
            negative_nodes.astype(jnp.int32).T,
        ],
        axis=0,
    )
    r = idx % TL
    sel = (r >= TL // 2).astype(jnp.float32)
    parity = jnp.zeros((PAR_ROWS, BATCH), jnp.float32)
    parity = parity.at[:NUM_ROWS].set(sel)
    half = (idx // TL) * (TL // 2) + (r % (TL // 2))
    idx_cn = (
        half[1:].reshape(NUM_CN, NW, BATCH // NW)
        .transpose(1, 0, 2).reshape(-1)
    )
    idx_c = half[0].reshape(NW, BATCH // NW).reshape(-1)
    out_wide = _relayout(output_emb.T)
    g_cn = _sc_gather(out_wide, idx_cn, NUM_CN)
    in_wide = _relayout(input_emb.T)
    g_center = _sc_gather(in_wide, idx_c, 1)
    return _tc_loss(g_center, g_cn, parity)


# speedup vs baseline: 1.9749x; 1.1562x over previous
import functools

import jax
import jax.numpy as jnp
from jax import lax
from jax.experimental import pallas as pl
from jax.experimental.pallas import tpu as pltpu
from jax.experimental.pallas import tpu_sc as plsc

VOCAB = 1000000
DIM = 64
WIDE = 128
BATCH = 16384
NUM_NEG = 10
NUM_ROWS = NUM_NEG + 2
NUM_CN = NUM_NEG + 1
PAR_ROWS = 16

NC = 2
NS = 16
NW = NC * NS
SUB = 256

TL = 16384
NWB = (VOCAB + TL - 1) // TL
WROWS = NWB * (TL // 2)


def _relayout_kernel(t_ref, o_ref):
    blk = t_ref[...]
    row = lax.broadcasted_iota(jnp.int32, (DIM, DIM), 0)
    col = lax.broadcasted_iota(jnp.int32, (DIM, DIM), 1)
    eye = (row == col).astype(jnp.float32)
    dn = (((0,), (0,)), ((), ()))
    o_ref[:, :DIM] = lax.dot_general(
        blk[:, : TL // 2], eye, dn, preferred_element_type=jnp.float32)
    o_ref[:, DIM:] = lax.dot_general(
        blk[:, TL // 2:], eye, dn, preferred_element_type=jnp.float32)


def _relayout(table_t):
    return pl.pallas_call(
        _relayout_kernel,
        grid=(NWB,),
        in_specs=[pl.BlockSpec((DIM, TL), lambda i: (0, i))],
        out_specs=pl.BlockSpec((TL // 2, WIDE), lambda i: (i, 0)),
        out_shape=jax.ShapeDtypeStruct((WROWS, WIDE), jnp.float32),
        compiler_params=pltpu.CompilerParams(
            dimension_semantics=("parallel",)),
    )(table_t)


def _sc_gather(table_wide, idx_flat, n_rows):
    per_w = n_rows * BATCH // NW
    nchunk = per_w // SUB
    chunk_b = BATCH // NW
    mesh = plsc.VectorSubcoreMesh(core_axis_name="c", subcore_axis_name="s")

    @functools.partial(
        pl.kernel,
        mesh=mesh,
        out_type=jax.ShapeDtypeStruct((n_rows * BATCH, WIDE), jnp.float32),
        scratch_types=[
            pltpu.VMEM((per_w,), jnp.int32),
            pltpu.VMEM((SUB, WIDE), jnp.float32),
            pltpu.VMEM((SUB, WIDE), jnp.float32),
            pltpu.SemaphoreType.DMA,
            pltpu.SemaphoreType.DMA,
        ],
    )
    def gather_kernel(t_hbm, idx_hbm, g_hbm, idx_v, rows_a, rows_b,
                      sem_a, sem_b):
        wid = lax.axis_index("s") * NC + lax.axis_index("c")
        pltpu.sync_copy(idx_hbm.at[pl.ds(wid * per_w, per_w)], idx_v)
        bufs = (rows_a, rows_b)
        sems = (sem_a, sem_b)
        pend = [None, None]

        def dst(c):
            j, h = divmod(c, chunk_b // SUB)
            return j * BATCH + wid * chunk_b + h * SUB

        for c in range(nchunk):
            s = c % 2
            pend[s] = pltpu.async_copy(
                t_hbm.at[idx_v.at[pl.ds(c * SUB, SUB)]], bufs[s], sems[s]
            )
            if c > 0:
                pend[1 - s].wait()
                pltpu.sync_copy(
                    bufs[1 - s], g_hbm.at[pl.ds(dst(c - 1), SUB)]
                )
        s = (nchunk - 1) % 2
        pend[s].wait()
        pltpu.sync_copy(bufs[s], g_hbm.at[pl.ds(dst(nchunk - 1), SUB)])

    return gather_kernel(table_wide, idx_flat)


BB = 1024


def _loss_kernel(c_ref, n_ref, p_ref, o_ref):
    i = pl.program_id(1)
    c = c_ref[...]
    n = n_ref[...]
    par = p_ref[...]
    csel = jnp.where(par[0][:, None] > 0.5, c[:, DIM:], c[:, :DIM])
    csel2 = jnp.concatenate([csel, csel], axis=-1)
    prod = csel2[None, :, :] * n
    lane = lax.broadcasted_iota(jnp.int32, (WIDE, 8), 0)
    col = lax.broadcasted_iota(jnp.int32, (WIDE, 8), 1)
    mask = ((col == (lane // DIM)) & (col < 2)).astype(jnp.float32)
    s2 = lax.dot_general(
        prod.reshape(NUM_CN * BB, WIDE), mask,
        (((1,), (0,)), ((), ())), preferred_element_type=jnp.float32,
    )
    s2t = s2.T
    s_lo, s_hi = s2t[0], s2t[1]
    pn = par[1:NUM_ROWS].reshape(NUM_CN * BB)
    scores = s_lo + pn * (s_hi - s_lo)

    def softplus(x):
        return jnp.maximum(x, 0.0) + jnp.log1p(jnp.exp(-jnp.abs(x)))

    block = (jnp.sum(softplus(-scores[:BB]))
             + jnp.sum(softplus(scores[BB:])))

    @pl.when(i == 0)
    def _():
        o_ref[...] = jnp.zeros_like(o_ref)

    o_ref[...] += block


def _tc_loss(g_center, g_cn, parity):
    n3 = g_cn.reshape(NUM_CN, BATCH, WIDE)
    nb2 = BATCH // BB // 2
    out = pl.pallas_call(
        _loss_kernel,
        grid=(2, nb2),
        in_specs=[
            pl.BlockSpec((BB, WIDE), lambda c, j: (c * nb2 + j, 0)),
            pl.BlockSpec((NUM_CN, BB, WIDE), lambda c, j: (0, c * nb2 + j, 0)),
            pl.BlockSpec((PAR_ROWS, BB), lambda c, j: (0, c * nb2 + j)),
        ],
        out_specs=pl.BlockSpec((1, 1, 1), lambda c, j: (c, 0, 0)),
        out_shape=jax.ShapeDtypeStruct((2, 1, 1), jnp.float32),
        compiler_params=pltpu.CompilerParams(
            dimension_semantics=("parallel", "arbitrary")),
    )(g_center, n3, parity)
    return (out[0, 0, 0] + out[1, 0, 0]) / BATCH


def kernel(center_nodes, context_nodes, negative_nodes, input_emb, output_emb):
    idx = jnp.concatenate(
        [
            center_nodes.astype(jnp.int32)[None, :],
            context_nodes.astype(jnp.int32)[None, :],
            negative_nodes.astype(jnp.int32).T,
        ],
        axis=0,
    )
    r = idx % TL
    sel = (r >= TL // 2).astype(jnp.float32)
    parity = jnp.zeros((PAR_ROWS, BATCH), jnp.float32)
    parity = parity.at[:NUM_ROWS].set(sel)
    half = (idx // TL) * (TL // 2) + (r % (TL // 2))
    idx_cn = (
        half[1:].reshape(NUM_CN, NW, BATCH // NW)
        .transpose(1, 0, 2).reshape(-1)
    )
    idx_c = half[0].reshape(NW, BATCH // NW).reshape(-1)
    out_wide = _relayout(output_emb.T)
    g_cn = _sc_gather(out_wide, idx_cn, NUM_CN)
    in_wide = _relayout(input_emb.T)
    g_center = _sc_gather(in_wide, idx_c, 1)
    return _tc_loss(g_center, g_cn, parity)


# speedup vs baseline: 2.1920x; 1.1100x over previous
import functools

import jax
import jax.numpy as jnp
from jax import lax
from jax.experimental import pallas as pl
from jax.experimental.pallas import tpu as pltpu
from jax.experimental.pallas import tpu_sc as plsc

VOCAB = 1000000
DIM = 64
WIDE = 128
BATCH = 16384
NUM_NEG = 10
NUM_ROWS = NUM_NEG + 2
NUM_CN = NUM_NEG + 1
PAR_ROWS = 16

NC = 2
NS = 16
NW = NC * NS
SUB = 256

TL = 16384
NWB = (VOCAB + TL - 1) // TL
TL4 = TL // 4
WROWS = NWB * TL4


def _relayout_kernel(t_ref, o_ref):
    blk = t_ref[...]
    row = lax.broadcasted_iota(jnp.int32, (DIM, DIM), 0)
    col = lax.broadcasted_iota(jnp.int32, (DIM, DIM), 1)
    eye = (row == col).astype(jnp.float32)
    dn = (((0,), (0,)), ((), ()))
    h0 = lax.dot_general(blk[:, : TL // 2], eye, dn,
                         preferred_element_type=jnp.float32)
    h1 = lax.dot_general(blk[:, TL // 2:], eye, dn,
                         preferred_element_type=jnp.float32)
    y = jnp.concatenate([h0, h1], axis=-1)
    o_ref[...] = pltpu.bitcast(y.astype(jnp.bfloat16), jnp.float32)


def _relayout(table_t):
    return pl.pallas_call(
        _relayout_kernel,
        grid=(NWB,),
        in_specs=[pl.BlockSpec((DIM, TL), lambda i: (0, i))],
        out_specs=pl.BlockSpec((TL4, WIDE), lambda i: (i, 0)),
        out_shape=jax.ShapeDtypeStruct((WROWS, WIDE), jnp.float32),
        compiler_params=pltpu.CompilerParams(
            dimension_semantics=("parallel",)),
    )(table_t)


def _sc_gather(table_wide, idx_flat, n_rows):
    per_w = n_rows * BATCH // NW
    nchunk = per_w // SUB
    chunk_b = BATCH // NW
    mesh = plsc.VectorSubcoreMesh(core_axis_name="c", subcore_axis_name="s")

    @functools.partial(
        pl.kernel,
        mesh=mesh,
        out_type=jax.ShapeDtypeStruct((n_rows * BATCH, WIDE), jnp.float32),
        scratch_types=[
            pltpu.VMEM((per_w,), jnp.int32),
            pltpu.VMEM((SUB, WIDE), jnp.float32),
            pltpu.VMEM((SUB, WIDE), jnp.float32),
            pltpu.SemaphoreType.DMA,
            pltpu.SemaphoreType.DMA,
        ],
    )
    def gather_kernel(t_hbm, idx_hbm, g_hbm, idx_v, rows_a, rows_b,
                      sem_a, sem_b):
        wid = lax.axis_index("s") * NC + lax.axis_index("c")
        pltpu.sync_copy(idx_hbm.at[pl.ds(wid * per_w, per_w)], idx_v)
        bufs = (rows_a, rows_b)
        sems = (sem_a, sem_b)
        pend = [None, None]

        def dst(c):
            j, h = divmod(c, chunk_b // SUB)
            return j * BATCH + wid * chunk_b + h * SUB

        for c in range(nchunk):
            s = c % 2
            pend[s] = pltpu.async_copy(
                t_hbm.at[idx_v.at[pl.ds(c * SUB, SUB)]], bufs[s], sems[s]
            )
            if c > 0:
                pend[1 - s].wait()
                pltpu.sync_copy(
                    bufs[1 - s], g_hbm.at[pl.ds(dst(c - 1), SUB)]
                )
        s = (nchunk - 1) % 2
        pend[s].wait()
        pltpu.sync_copy(bufs[s], g_hbm.at[pl.ds(dst(nchunk - 1), SUB)])

    return gather_kernel(table_wide, idx_flat)


BB = 1024


def _unpack(x, code):
    xu = lax.bitcast_convert_type(x, jnp.uint32)
    p = (code % 2)[..., None]
    bits = jnp.where(p == 1, xu & jnp.uint32(0xFFFF0000), xu << 16)
    return lax.bitcast_convert_type(bits, jnp.float32)


def _loss_kernel(c_ref, n_ref, p_ref, o_ref):
    i = pl.program_id(1)
    c = c_ref[...]
    n = n_ref[...]
    par = p_ref[...]
    c128 = _unpack(c, par[0])
    csel = jnp.where(par[0][:, None] >= 2, c128[:, DIM:], c128[:, :DIM])
    csel2 = jnp.concatenate([csel, csel], axis=-1)
    rows = _unpack(n, par[1:NUM_ROWS])
    prod = csel2[None, :, :] * rows
    lane = lax.broadcasted_iota(jnp.int32, (WIDE, 8), 0)
    col = lax.broadcasted_iota(jnp.int32, (WIDE, 8), 1)
    mask = ((col == (lane // DIM)) & (col < 2)).astype(jnp.float32)
    s2 = lax.dot_general(
        prod.reshape(NUM_CN * BB, WIDE), mask,
        (((1,), (0,)), ((), ())), preferred_element_type=jnp.float32,
    )
    s2t = s2.T
    s_lo, s_hi = s2t[0], s2t[1]
    hn = (par[1:NUM_ROWS].reshape(NUM_CN * BB) >= 2).astype(jnp.float32)
    scores = s_lo + hn * (s_hi - s_lo)

    def softplus(x):
        return jnp.maximum(x, 0.0) + jnp.log1p(jnp.exp(-jnp.abs(x)))

    block = (jnp.sum(softplus(-scores[:BB]))
             + jnp.sum(softplus(scores[BB:])))

    @pl.when(i == 0)
    def _():
        o_ref[...] = jnp.zeros_like(o_ref)

    o_ref[...] += block


def _tc_loss(g_center, g_cn, parity):
    n3 = g_cn.reshape(NUM_CN, BATCH, WIDE)
    nb2 = BATCH // BB // 2
    out = pl.pallas_call(
        _loss_kernel,
        grid=(2, nb2),
        in_specs=[
            pl.BlockSpec((BB, WIDE), lambda c, j: (c * nb2 + j, 0)),
            pl.BlockSpec((NUM_CN, BB, WIDE), lambda c, j: (0, c * nb2 + j, 0)),
            pl.BlockSpec((PAR_ROWS, BB), lambda c, j: (0, c * nb2 + j)),
        ],
        out_specs=pl.BlockSpec((1, 1, 1), lambda c, j: (c, 0, 0)),
        out_shape=jax.ShapeDtypeStruct((2, 1, 1), jnp.float32),
        compiler_params=pltpu.CompilerParams(
            dimension_semantics=("parallel", "arbitrary")),
    )(g_center, n3, parity)
    return (out[0, 0, 0] + out[1, 0, 0]) / BATCH


def kernel(center_nodes, context_nodes, negative_nodes, input_emb, output_emb):
    idx = jnp.concatenate(
        [
            center_nodes.astype(jnp.int32)[None, :],
            context_nodes.astype(jnp.int32)[None, :],
            negative_nodes.astype(jnp.int32).T,
        ],
        axis=0,
    )
    rr = idx % TL
    h = rr // (TL // 2)
    u = rr % (TL // 2)
    code = (2 * h + (u % 2)).astype(jnp.float32)
    parity = jnp.zeros((PAR_ROWS, BATCH), jnp.float32)
    parity = parity.at[:NUM_ROWS].set(code)
    half = (idx // TL) * TL4 + (u // 2)
    idx_cn = (
        half[1:].reshape(NUM_CN, NW, BATCH // NW)
        .transpose(1, 0, 2).reshape(-1)
    )
    idx_c = half[0].reshape(NW, BATCH // NW).reshape(-1)
    out_wide = _relayout(output_emb.T)
    g_cn = _sc_gather(out_wide, idx_cn, NUM_CN)
    in_wide = _relayout(input_emb.T)
    g_center = _sc_gather(in_wide, idx_c, 1)
    return _tc_loss(g_center, g_cn, parity)


# speedup vs baseline: 2.2759x; 1.0383x over previous
import functools

import jax
import jax.numpy as jnp
from jax import lax
from jax.experimental import pallas as pl
from jax.experimental.pallas import tpu as pltpu
from jax.experimental.pallas import tpu_sc as plsc

VOCAB = 1000000
DIM = 64
WIDE = 128
BATCH = 16384
NUM_NEG = 10
NUM_ROWS = NUM_NEG + 2
NUM_CN = NUM_NEG + 1
PAR_ROWS = 16

NC = 2
NS = 16
NW = NC * NS
SUB = 256

TL = 24576
NWB = (VOCAB + TL - 1) // TL
TL4 = TL // 4
WROWS = NWB * TL4


def _relayout_kernel(t_ref, o_ref):
    blk = t_ref[...]
    row = lax.broadcasted_iota(jnp.int32, (DIM, DIM), 0)
    col = lax.broadcasted_iota(jnp.int32, (DIM, DIM), 1)
    eye = (row == col).astype(jnp.float32)
    dn = (((0,), (0,)), ((), ()))
    h0 = lax.dot_general(blk[:, : TL // 2], eye, dn,
                         preferred_element_type=jnp.float32)
    h1 = lax.dot_general(blk[:, TL // 2:], eye, dn,
                         preferred_element_type=jnp.float32)
    y = jnp.concatenate([h0, h1], axis=-1)
    o_ref[...] = pltpu.bitcast(y.astype(jnp.bfloat16), jnp.float32)


def _relayout(table_t):
    return pl.pallas_call(
        _relayout_kernel,
        grid=(NWB,),
        in_specs=[pl.BlockSpec((DIM, TL), lambda i: (0, i))],
        out_specs=pl.BlockSpec((TL4, WIDE), lambda i: (i, 0)),
        out_shape=jax.ShapeDtypeStruct((WROWS, WIDE), jnp.float32),
        compiler_params=pltpu.CompilerParams(
            dimension_semantics=("parallel",)),
    )(table_t)


def _sc_gather(table_wide, idx_flat, n_rows):
    per_w = n_rows * BATCH // NW
    nchunk = per_w // SUB
    chunk_b = BATCH // NW
    mesh = plsc.VectorSubcoreMesh(core_axis_name="c", subcore_axis_name="s")

    @functools.partial(
        pl.kernel,
        mesh=mesh,
        out_type=jax.ShapeDtypeStruct((n_rows * BATCH, WIDE), jnp.float32),
        scratch_types=[
            pltpu.VMEM((per_w,), jnp.int32),
            pltpu.VMEM((SUB, WIDE), jnp.float32),
            pltpu.VMEM((SUB, WIDE), jnp.float32),
            pltpu.SemaphoreType.DMA,
            pltpu.SemaphoreType.DMA,
        ],
    )
    def gather_kernel(t_hbm, idx_hbm, g_hbm, idx_v, rows_a, rows_b,
                      sem_a, sem_b):
        wid = lax.axis_index("s") * NC + lax.axis_index("c")
        pltpu.sync_copy(idx_hbm.at[pl.ds(wid * per_w, per_w)], idx_v)
        bufs = (rows_a, rows_b)
        sems = (sem_a, sem_b)
        pend = [None, None]

        def dst(c):
            j, h = divmod(c, chunk_b // SUB)
            return j * BATCH + wid * chunk_b + h * SUB

        for c in range(nchunk):
            s = c % 2
            pend[s] = pltpu.async_copy(
                t_hbm.at[idx_v.at[pl.ds(c * SUB, SUB)]], bufs[s], sems[s]
            )
            if c > 0:
                pend[1 - s].wait()
                pltpu.sync_copy(
                    bufs[1 - s], g_hbm.at[pl.ds(dst(c - 1), SUB)]
                )
        s = (nchunk - 1) % 2
        pend[s].wait()
        pltpu.sync_copy(bufs[s], g_hbm.at[pl.ds(dst(nchunk - 1), SUB)])

    return gather_kernel(table_wide, idx_flat)


BB = 1024


def _unpack(x, code):
    xu = lax.bitcast_convert_type(x, jnp.uint32)
    p = (code % 2)[..., None]
    bits = jnp.where(p == 1, xu & jnp.uint32(0xFFFF0000), xu << 16)
    return lax.bitcast_convert_type(bits, jnp.float32)


def _loss_kernel(c_ref, n_ref, p_ref, o_ref):
    i = pl.program_id(1)
    c = c_ref[...]
    n = n_ref[...]
    par = p_ref[...]
    c128 = _unpack(c, par[0])
    csel = jnp.where(par[0][:, None] >= 2, c128[:, DIM:], c128[:, :DIM])
    csel2 = jnp.concatenate([csel, csel], axis=-1)
    rows = _unpack(n, par[1:NUM_ROWS])
    prod = csel2[None, :, :] * rows
    lane = lax.broadcasted_iota(jnp.int32, (WIDE, 8), 0)
    col = lax.broadcasted_iota(jnp.int32, (WIDE, 8), 1)
    mask = ((col == (lane // DIM)) & (col < 2)).astype(jnp.float32)
    s2 = lax.dot_general(
        prod.reshape(NUM_CN * BB, WIDE), mask,
        (((1,), (0,)), ((), ())), preferred_element_type=jnp.float32,
    )
    s2t = s2.T
    s_lo, s_hi = s2t[0], s2t[1]
    hn = (par[1:NUM_ROWS].reshape(NUM_CN * BB) >= 2).astype(jnp.float32)
    scores = s_lo + hn * (s_hi - s_lo)

    def softplus(x):
        return jnp.maximum(x, 0.0) + jnp.log1p(jnp.exp(-jnp.abs(x)))

    block = (jnp.sum(softplus(-scores[:BB]))
             + jnp.sum(softplus(scores[BB:])))

    @pl.when(i == 0)
    def _():
        o_ref[...] = jnp.zeros_like(o_ref)

    o_ref[...] += block


def _tc_loss(g_center, g_cn, parity):
    n3 = g_cn.reshape(NUM_CN, BATCH, WIDE)
    nb2 = BATCH // BB // 2
    out = pl.pallas_call(
        _loss_kernel,
        grid=(2, nb2),
        in_specs=[
            pl.BlockSpec((BB, WIDE), lambda c, j: (c * nb2 + j, 0)),
            pl.BlockSpec((NUM_CN, BB, WIDE), lambda c, j: (0, c * nb2 + j, 0)),
            pl.BlockSpec((PAR_ROWS, BB), lambda c, j: (0, c * nb2 + j)),
        ],
        out_specs=pl.BlockSpec((1, 1, 1), lambda c, j: (c, 0, 0)),
        out_shape=jax.ShapeDtypeStruct((2, 1, 1), jnp.float32),
        compiler_params=pltpu.CompilerParams(
            dimension_semantics=("parallel", "arbitrary")),
    )(g_center, n3, parity)
    return (out[0, 0, 0] + out[1, 0, 0]) / BATCH


def kernel(center_nodes, context_nodes, negative_nodes, input_emb, output_emb):
    idx = jnp.concatenate(
        [
            center_nodes.astype(jnp.int32)[None, :],
            context_nodes.astype(jnp.int32)[None, :],
            negative_nodes.astype(jnp.int32).T,
        ],
        axis=0,
    )
    rr = idx % TL
    h = rr // (TL // 2)
    u = rr % (TL // 2)
    code = (2 * h + (u % 2)).astype(jnp.float32)
    parity = jnp.zeros((PAR_ROWS, BATCH), jnp.float32)
    parity = parity.at[:NUM_ROWS].set(code)
    half = (idx // TL) * TL4 + (u // 2)
    idx_cn = (
        half[1:].reshape(NUM_CN, NW, BATCH // NW)
        .transpose(1, 0, 2).reshape(-1)
    )
    idx_c = half[0].reshape(NW, BATCH // NW).reshape(-1)
    out_wide = _relayout(output_emb.T)
    g_cn = _sc_gather(out_wide, idx_cn, NUM_CN)
    in_wide = _relayout(input_emb.T)
    g_center = _sc_gather(in_wide, idx_c, 1)
    return _tc_loss(g_center, g_cn, parity)
